# Initial kernel scaffold; baseline (speedup 1.0000x reference)
import jax
import jax.numpy as jnp
from jax.experimental import pallas as pl


def kernel(x, edge_index, hidden_states, cell_states, params):
    raise NotImplementedError("write your pallas kernel here")


# speedup vs baseline: 5.6361x; 5.6361x over previous
import functools

import jax
import jax.numpy as jnp
from jax import lax
from jax.experimental import pallas as pl
from jax.experimental.pallas import tpu as pltpu
from jax.experimental.pallas import tpu_sc as plsc

N = 10000
E = 160000
F = 256
FH = 128
NSUB = 16
SEQ = 6
NLAYERS = 2

EPW = E // NSUB
CH = 80
NBUF = 4
NCH = EPW // CH
WB = 80
NWB = N // WB
WBROUNDS = -(-NWB // NSUB)


_sc_mesh = plsc.VectorSubcoreMesh(core_axis_name="c", subcore_axis_name="s")


@functools.partial(
    pl.kernel,
    mesh=_sc_mesh,
    out_type=jax.ShapeDtypeStruct((2, N, FH), jnp.float32),
    scratch_types=(
        [pltpu.VMEM((CH,), jnp.int32) for _ in range(NBUF)]
        + [pltpu.VMEM((CH,), jnp.int32) for _ in range(NBUF)]
        + [pltpu.VMEM((CH, FH), jnp.float32) for _ in range(NBUF)]
        + [pltpu.VMEM_SHARED((N, FH), jnp.float32)]
        + [pltpu.SemaphoreType.DMA for _ in range(NBUF)]
    ),
)
def _sc_agg(feat2, src, dst, out,
            ig0, ig1, ig2, ig3,
            id0, id1, id2, id3,
            rb0, rb1, rb2, rb3,
            acc, sg0, sg1, sg2, sg3):
    c = lax.axis_index("c")
    s = lax.axis_index("s")
    igs = (ig0, ig1, ig2, ig3)
    ids = (id0, id1, id2, id3)
    rbs = (rb0, rb1, rb2, rb3)
    sgs = (sg0, sg1, sg2, sg3)

    zv = jnp.zeros((16,), jnp.float32)

    def zloop(r, carry):
        for jj in range(FH // 16):
            rb0[r, pl.ds(jj * 16, 16)] = zv
        return carry

    lax.fori_loop(0, WB, zloop, 0)
    for t in range(WBROUNDS):
        k = s + t * NSUB

        @pl.when(k < NWB)
        def _():
            pltpu.sync_copy(rb0, acc.at[pl.ds(pl.multiple_of(k * WB, 8), WB)])

    plsc.subcore_barrier()

    base_e = s * EPW

    def load_fire(k, j):
        off = pl.multiple_of(base_e + k * CH, 8)
        pltpu.sync_copy(src.at[pl.ds(off, CH)], igs[j])
        pltpu.sync_copy(dst.at[pl.ds(off, CH)], ids[j])
        for t in range(CH // 16):
            sl = pl.ds(t * 16, 16)
            igs[j][sl] = igs[j][sl] * 2 + c
        pltpu.make_async_copy(feat2.at[igs[j]], rbs[j], sgs[j]).start()

    def drain_scatter(j):
        pltpu.make_async_copy(feat2.at[igs[j]], rbs[j], sgs[j]).wait()
        pltpu.sync_copy(rbs[j], acc.at[ids[j]], add=True)

    for j in range(NBUF):
        load_fire(j, j)

    def mainblk(kb, carry):
        for j in range(NBUF):
            k = kb * NBUF + j
            drain_scatter(j)

            @pl.when(k + NBUF < NCH)
            def _():
                load_fire(k + NBUF, j)

        return carry

    lax.fori_loop(0, NCH // NBUF, mainblk, 0)
    drain_scatter((NCH - 1) % NBUF)

    plsc.subcore_barrier()
    for t in range(WBROUNDS):
        k = s + t * NSUB

        @pl.when(k < NWB)
        def _():
            r0 = pl.multiple_of(k * WB, 8)
            pltpu.sync_copy(acc.at[pl.ds(r0, WB)], rb0)
            pltpu.sync_copy(rb0, out.at[c, pl.ds(r0, WB)])


def _agg(feat, src, dst):
    return _sc_agg(feat.reshape(2 * N, FH), src, dst)


BT = 2000


def _gates_body(ax0, ax1, ah0, ah1, deg, h, w, b, rh_o, u_o):
    dinv = 1.0 / jnp.maximum(deg[0], 1.0)
    a = jnp.concatenate(
        [ax0[0] * dinv, ax1[0] * dinv, ah0[0] * dinv, ah1[0] * dinv], axis=1)
    pre = jnp.dot(a, w[...], preferred_element_type=jnp.float32) + b[...]
    r = jax.nn.sigmoid(pre[:, :F])
    u = jax.nn.sigmoid(pre[:, F:])
    rh_o[...] = r * h[...]
    u_o[...] = u


def _combine_body(ax0, ax1, ar0, ar1, deg, h, u, w, b, h_o, c_o):
    dinv = 1.0 / jnp.maximum(deg[0], 1.0)
    a = jnp.concatenate(
        [ax0[0] * dinv, ax1[0] * dinv, ar0[0] * dinv, ar1[0] * dinv], axis=1)
    cc = jnp.tanh(jnp.dot(a, w[...], preferred_element_type=jnp.float32) + b[...])
    uu = u[...]
    h_o[...] = uu * h[...] + (1.0 - uu) * cc
    c_o[...] = cc


def _half_spec(which):
    return pl.BlockSpec((1, BT, FH), lambda i, w=which: (w, i, 0))


def _row_spec():
    return pl.BlockSpec((BT, F), lambda i: (i, 0))


_gates = pl.pallas_call(
    _gates_body,
    grid=(N // BT,),
    in_specs=[
        _half_spec(0), _half_spec(1), _half_spec(0), _half_spec(1), _half_spec(0),
        _row_spec(),
        pl.BlockSpec((2 * F, 2 * F), lambda i: (0, 0)),
        pl.BlockSpec((1, 2 * F), lambda i: (0, 0)),
    ],
    out_specs=[_row_spec(), _row_spec()],
    out_shape=[jax.ShapeDtypeStruct((N, F), jnp.float32)] * 2,
)

_combine = pl.pallas_call(
    _combine_body,
    grid=(N // BT,),
    in_specs=[
        _half_spec(0), _half_spec(1), _half_spec(0), _half_spec(1), _half_spec(0),
        _row_spec(), _row_spec(),
        pl.BlockSpec((2 * F, F), lambda i: (0, 0)),
        pl.BlockSpec((1, F), lambda i: (0, 0)),
    ],
    out_specs=[_row_spec(), _row_spec()],
    out_shape=[jax.ShapeDtypeStruct((N, F), jnp.float32)] * 2,
)


def _proj_body(z, w, b, o):
    o[...] = jnp.dot(z[...], w[...], preferred_element_type=jnp.float32) + b[...]


_proj = pl.pallas_call(
    _proj_body,
    grid=(SEQ * N // BT,),
    in_specs=[
        pl.BlockSpec((BT, F), lambda i: (i, 0)),
        pl.BlockSpec((F, F), lambda i: (0, 0)),
        pl.BlockSpec((1, F), lambda i: (0, 0)),
    ],
    out_specs=pl.BlockSpec((BT, F), lambda i: (i, 0)),
    out_shape=jax.ShapeDtypeStruct((SEQ * N, F), jnp.float32),
)


def kernel(x, edge_index, hidden_states, cell_states, params):
    src = edge_index[0].astype(jnp.int32)
    dst = edge_index[1].astype(jnp.int32)

    deg2 = _agg(jnp.ones((N, F), jnp.float32), src, dst)

    layers = params['layers']
    w_ru = [jnp.concatenate([p['Wr'], p['Wu']], axis=1) for p in layers]
    b_ru = [jnp.concatenate([p['br'], p['bu']])[None, :] for p in layers]
    w_c = [p['Wc'] for p in layers]
    b_c = [p['bc'][None, :] for p in layers]

    h = [hidden_states[j] for j in range(NLAYERS)]
    c = [None] * NLAYERS
    outs = []
    for i in range(SEQ):
        inp = x[i]
        for j in range(NLAYERS):
            aggx = _agg(inp, src, dst)
            aggh = _agg(h[j], src, dst)
            rh, u = _gates(aggx, aggx, aggh, aggh, deg2, h[j], w_ru[j], b_ru[j])
            aggr = _agg(rh, src, dst)
            hnew, cnew = _combine(aggx, aggx, aggr, aggr, deg2, h[j], u,
                                  w_c[j], b_c[j])
            h[j] = hnew
            c[j] = cnew
            inp = hnew
        outs.append(inp)

    z = jnp.stack(outs, 0).reshape(SEQ * N, F)
    out = _proj(z, params['Wo'], params['bo'][None, :]).reshape(SEQ, N, F)
    return out, jnp.stack(h, 0), jnp.stack(c, 0)


# speedup vs baseline: 9.5105x; 1.6874x over previous
import functools

import jax
import jax.numpy as jnp
from jax import lax
from jax.experimental import pallas as pl
from jax.experimental.pallas import tpu as pltpu
from jax.experimental.pallas import tpu_sc as plsc

N = 10000
E = 160000
F = 256
FH = 128
NSUB = 16
SEQ = 6
NLAYERS = 2

EPW = E // NSUB
CH = 80
NBUF = 3
NCH = EPW // CH
WB = 80
NWB = N // WB
WBROUNDS = -(-NWB // NSUB)
GRP = 25
NGRP = NCH // GRP


_sc_mesh = plsc.VectorSubcoreMesh(core_axis_name="c", subcore_axis_name="s")


@functools.partial(
    pl.kernel,
    mesh=_sc_mesh,
    out_type=jax.ShapeDtypeStruct((2, N, FH), jnp.float32),
    scratch_types=(
        [pltpu.VMEM((EPW,), jnp.int32)]
        + [pltpu.VMEM((2, GRP, CH), jnp.int32)]
        + [pltpu.VMEM((CH, FH), jnp.float32) for _ in range(NBUF)]
        + [pltpu.VMEM_SHARED((N, FH), jnp.float32)]
        + [pltpu.SemaphoreType.DMA for _ in range(2 * NBUF + 1)]
    ),
)
def _sc_agg(feat2, gidx, sidx, out,
            srcb, dstb, rb0, rb1, rb2,
            acc, sg0, sg1, sg2, ss0, ss1, ss2, si):
    c = lax.axis_index("c")
    s = lax.axis_index("s")
    rbs = (rb0, rb1, rb2)
    sgs = (sg0, sg1, sg2)
    sss = (ss0, ss1, ss2)

    pltpu.sync_copy(gidx.at[c, s], srcb)
    pltpu.sync_copy(sidx.at[s, 0], dstb.at[0])

    zv = jnp.zeros((16,), jnp.float32)

    def zloop(r, carry):
        for jj in range(FH // 16):
            rb0[r, pl.ds(jj * 16, 16)] = zv
        return carry

    lax.fori_loop(0, WB, zloop, 0)
    for t in range(WBROUNDS):
        k = s + t * NSUB

        @pl.when(k < NWB)
        def _():
            pltpu.sync_copy(rb0, acc.at[pl.ds(pl.multiple_of(k * WB, 8), WB)])

    plsc.subcore_barrier()

    def grow(k):
        return acc.at[dstb.at[lax.rem(lax.div(k, GRP), 2), lax.rem(k, GRP)]]

    def fire_gather(k, j):
        idx = srcb.at[pl.ds(pl.multiple_of(k * CH, 8), CH)]
        pltpu.make_async_copy(feat2.at[idx], rbs[j], sgs[j]).start()

    def wait_gather(k, j):
        idx = srcb.at[pl.ds(pl.multiple_of(k * CH, 8), CH)]
        pltpu.make_async_copy(feat2.at[idx], rbs[j], sgs[j]).wait()

    def step(k, j):
        jn = (j + 1) % NBUF
        kg = lax.rem(k, GRP)

        @pl.when(jnp.logical_and(kg == 0, k > 0))
        def _():
            pltpu.make_async_copy(sidx.at[s, 0], dstb.at[0], si).wait()

        @pl.when(k >= 2)
        def _():
            pltpu.make_async_copy(rbs[jn], grow(k), sss[jn]).wait()

        @pl.when(jnp.logical_and(kg == 2, k < (NGRP - 1) * GRP))
        def _():
            g1 = lax.div(k, GRP) + 1
            pltpu.make_async_copy(sidx.at[s, g1], dstb.at[lax.rem(g1, 2)],
                                  si).start()

        @pl.when(k + 1 < NCH)
        def _():
            fire_gather(k + 1, jn)

        wait_gather(k, j)
        pltpu.async_copy(rbs[j], grow(k), sss[j], add=True)

    fire_gather(0, 0)

    def mainblk(kb, carry):
        for j in range(NBUF):
            step(kb * NBUF + j, j)
        return carry

    lax.fori_loop(0, NCH // NBUF, mainblk, 0)
    pltpu.make_async_copy(rbs[1], grow(NCH - 4), sss[1]).wait()
    fire_gather(NCH - 1, 1)
    wait_gather(NCH - 2, 0)
    pltpu.async_copy(rbs[0], grow(NCH - 2), sss[0], add=True)
    pltpu.make_async_copy(rbs[2], grow(NCH - 3), sss[2]).wait()
    wait_gather(NCH - 1, 1)
    pltpu.async_copy(rbs[1], grow(NCH - 1), sss[1], add=True)
    pltpu.make_async_copy(rbs[0], grow(NCH - 2), sss[0]).wait()
    pltpu.make_async_copy(rbs[1], grow(NCH - 1), sss[1]).wait()

    plsc.subcore_barrier()
    for t in range(WBROUNDS):
        k = s + t * NSUB

        @pl.when(k < NWB)
        def _():
            r0 = pl.multiple_of(k * WB, 8)
            pltpu.sync_copy(acc.at[pl.ds(r0, WB)], rb0)
            pltpu.sync_copy(rb0, out.at[c, pl.ds(r0, WB)])


def _agg(feat, gidx, sidx):
    return _sc_agg(feat.reshape(2 * N, FH), gidx, sidx)


BT = 2000


def _gates_body(ax0, ax1, ah0, ah1, deg, h, w, b, rh_o, u_o):
    dinv = 1.0 / jnp.maximum(deg[0], 1.0)
    a = jnp.concatenate(
        [ax0[0] * dinv, ax1[0] * dinv, ah0[0] * dinv, ah1[0] * dinv], axis=1)
    pre = jnp.dot(a, w[...], preferred_element_type=jnp.float32) + b[...]
    r = jax.nn.sigmoid(pre[:, :F])
    u = jax.nn.sigmoid(pre[:, F:])
    rh_o[...] = r * h[...]
    u_o[...] = u


def _combine_body(ax0, ax1, ar0, ar1, deg, h, u, w, b, h_o, c_o):
    dinv = 1.0 / jnp.maximum(deg[0], 1.0)
    a = jnp.concatenate(
        [ax0[0] * dinv, ax1[0] * dinv, ar0[0] * dinv, ar1[0] * dinv], axis=1)
    cc = jnp.tanh(jnp.dot(a, w[...], preferred_element_type=jnp.float32) + b[...])
    uu = u[...]
    h_o[...] = uu * h[...] + (1.0 - uu) * cc
    c_o[...] = cc


def _half_spec(which):
    return pl.BlockSpec((1, BT, FH), lambda i, w=which: (w, i, 0))


def _row_spec():
    return pl.BlockSpec((BT, F), lambda i: (i, 0))


_gates = pl.pallas_call(
    _gates_body,
    grid=(N // BT,),
    in_specs=[
        _half_spec(0), _half_spec(1), _half_spec(0), _half_spec(1), _half_spec(0),
        _row_spec(),
        pl.BlockSpec((2 * F, 2 * F), lambda i: (0, 0)),
        pl.BlockSpec((1, 2 * F), lambda i: (0, 0)),
    ],
    out_specs=[_row_spec(), _row_spec()],
    out_shape=[jax.ShapeDtypeStruct((N, F), jnp.float32)] * 2,
)

_combine = pl.pallas_call(
    _combine_body,
    grid=(N // BT,),
    in_specs=[
        _half_spec(0), _half_spec(1), _half_spec(0), _half_spec(1), _half_spec(0),
        _row_spec(), _row_spec(),
        pl.BlockSpec((2 * F, F), lambda i: (0, 0)),
        pl.BlockSpec((1, F), lambda i: (0, 0)),
    ],
    out_specs=[_row_spec(), _row_spec()],
    out_shape=[jax.ShapeDtypeStruct((N, F), jnp.float32)] * 2,
)


def _proj_body(z, w, b, o):
    o[...] = jnp.dot(z[...], w[...], preferred_element_type=jnp.float32) + b[...]


_proj = pl.pallas_call(
    _proj_body,
    grid=(SEQ * N // BT,),
    in_specs=[
        pl.BlockSpec((BT, F), lambda i: (i, 0)),
        pl.BlockSpec((F, F), lambda i: (0, 0)),
        pl.BlockSpec((1, F), lambda i: (0, 0)),
    ],
    out_specs=pl.BlockSpec((BT, F), lambda i: (i, 0)),
    out_shape=jax.ShapeDtypeStruct((SEQ * N, F), jnp.float32),
)


def kernel(x, edge_index, hidden_states, cell_states, params):
    src = edge_index[0].astype(jnp.int32)
    dst = edge_index[1].astype(jnp.int32)
    gidx = jnp.stack([src * 2, src * 2 + 1]).reshape(2, NSUB, EPW)
    sidx = dst.reshape(NSUB, NGRP, GRP, CH)

    deg2 = _agg(jnp.ones((N, F), jnp.float32), gidx, sidx)

    layers = params['layers']
    w_ru = [jnp.concatenate([p['Wr'], p['Wu']], axis=1) for p in layers]
    b_ru = [jnp.concatenate([p['br'], p['bu']])[None, :] for p in layers]
    w_c = [p['Wc'] for p in layers]
    b_c = [p['bc'][None, :] for p in layers]

    h = [hidden_states[j] for j in range(NLAYERS)]
    c = [None] * NLAYERS
    outs = []
    for i in range(SEQ):
        inp = x[i]
        for j in range(NLAYERS):
            aggx = _agg(inp, gidx, sidx)
            aggh = _agg(h[j], gidx, sidx)
            rh, u = _gates(aggx, aggx, aggh, aggh, deg2, h[j], w_ru[j], b_ru[j])
            aggr = _agg(rh, gidx, sidx)
            hnew, cnew = _combine(aggx, aggx, aggr, aggr, deg2, h[j], u,
                                  w_c[j], b_c[j])
            h[j] = hnew
            c[j] = cnew
            inp = hnew
        outs.append(inp)

    z = jnp.stack(outs, 0).reshape(SEQ * N, F)
    out = _proj(z, params['Wo'], params['bo'][None, :]).reshape(SEQ, N, F)
    return out, jnp.stack(h, 0), jnp.stack(c, 0)


# speedup vs baseline: 9.6108x; 1.0105x over previous
import functools

import jax
import jax.numpy as jnp
from jax import lax
from jax.experimental import pallas as pl
from jax.experimental.pallas import tpu as pltpu
from jax.experimental.pallas import tpu_sc as plsc

N = 10000
E = 160000
F = 256
FH = 128
NSUB = 16
SEQ = 6
NLAYERS = 2

EPW = E // NSUB
CH = 80
NBUF = 3
NCH = EPW // CH
WB = 80
NWB = N // WB
WBROUNDS = -(-NWB // NSUB)
GRP = 25
NGRP = NCH // GRP


_sc_mesh = plsc.VectorSubcoreMesh(core_axis_name="c", subcore_axis_name="s")


@functools.partial(
    pl.kernel,
    mesh=_sc_mesh,
    out_type=jax.ShapeDtypeStruct((2, N, FH), jnp.float32),
    scratch_types=(
        [pltpu.VMEM((EPW,), jnp.int32)]
        + [pltpu.VMEM((2, GRP, CH), jnp.int32)]
        + [pltpu.VMEM((CH, FH), jnp.float32) for _ in range(NBUF)]
        + [pltpu.VMEM_SHARED((N, FH), jnp.float32)]
        + [pltpu.SemaphoreType.DMA for _ in range(2 * NBUF + 1)]
    ),
)
def _sc_agg(feat2, gidx, sidx, out,
            srcb, dstb, rb0, rb1, rb2,
            acc, sg0, sg1, sg2, ss0, ss1, ss2, si):
    c = lax.axis_index("c")
    s = lax.axis_index("s")
    rbs = (rb0, rb1, rb2)
    sgs = (sg0, sg1, sg2)
    sss = (ss0, ss1, ss2)

    pltpu.sync_copy(gidx.at[c, s], srcb)
    pltpu.sync_copy(sidx.at[s, 0], dstb.at[0])

    zv = jnp.zeros((16,), jnp.float32)

    def zloop(r, carry):
        for jj in range(FH // 16):
            rb0[r, pl.ds(jj * 16, 16)] = zv
        return carry

    lax.fori_loop(0, WB, zloop, 0)
    for t in range(WBROUNDS):
        k = s + t * NSUB

        @pl.when(k < NWB)
        def _():
            pltpu.sync_copy(rb0, acc.at[pl.ds(pl.multiple_of(k * WB, 8), WB)])

    plsc.subcore_barrier()

    def grow(k):
        return acc.at[dstb.at[lax.rem(lax.div(k, GRP), 2), lax.rem(k, GRP)]]

    def fire_gather(k, j):
        idx = srcb.at[pl.ds(pl.multiple_of(k * CH, 8), CH)]
        pltpu.make_async_copy(feat2.at[idx], rbs[j], sgs[j]).start()

    def wait_gather(k, j):
        idx = srcb.at[pl.ds(pl.multiple_of(k * CH, 8), CH)]
        pltpu.make_async_copy(feat2.at[idx], rbs[j], sgs[j]).wait()

    def step(k, j):
        jn = (j + 1) % NBUF
        kg = lax.rem(k, GRP)

        @pl.when(jnp.logical_and(kg == 0, k > 0))
        def _():
            pltpu.make_async_copy(sidx.at[s, 0], dstb.at[0], si).wait()

        @pl.when(k >= 2)
        def _():
            pltpu.make_async_copy(rbs[jn], grow(k), sss[jn]).wait()

        @pl.when(jnp.logical_and(kg == 2, k < (NGRP - 1) * GRP))
        def _():
            g1 = lax.div(k, GRP) + 1
            pltpu.make_async_copy(sidx.at[s, g1], dstb.at[lax.rem(g1, 2)],
                                  si).start()

        @pl.when(k + 1 < NCH)
        def _():
            fire_gather(k + 1, jn)

        wait_gather(k, j)
        pltpu.async_copy(rbs[j], grow(k), sss[j], add=True)

    fire_gather(0, 0)

    def mainblk(kb, carry):
        for j in range(NBUF):
            step(kb * NBUF + j, j)
        return carry

    lax.fori_loop(0, NCH // NBUF, mainblk, 0)
    pltpu.make_async_copy(rbs[1], grow(NCH - 4), sss[1]).wait()
    fire_gather(NCH - 1, 1)
    wait_gather(NCH - 2, 0)
    pltpu.async_copy(rbs[0], grow(NCH - 2), sss[0], add=True)
    pltpu.make_async_copy(rbs[2], grow(NCH - 3), sss[2]).wait()
    wait_gather(NCH - 1, 1)
    pltpu.async_copy(rbs[1], grow(NCH - 1), sss[1], add=True)
    pltpu.make_async_copy(rbs[0], grow(NCH - 2), sss[0]).wait()
    pltpu.make_async_copy(rbs[1], grow(NCH - 1), sss[1]).wait()

    plsc.subcore_barrier()
    for t in range(WBROUNDS):
        k = s + t * NSUB

        @pl.when(k < NWB)
        def _():
            r0 = pl.multiple_of(k * WB, 8)
            pltpu.sync_copy(acc.at[pl.ds(r0, WB)], out.at[c, pl.ds(r0, WB)])


def _agg(feat, gidx, sidx):
    return _sc_agg(feat.reshape(2 * N, FH), gidx, sidx)


BT = 2000


def _gates_body(ax0, ax1, ah0, ah1, deg, h, w, b, rh_o, u_o):
    dinv = 1.0 / jnp.maximum(deg[0], 1.0)
    a = jnp.concatenate(
        [ax0[0] * dinv, ax1[0] * dinv, ah0[0] * dinv, ah1[0] * dinv], axis=1)
    pre = jnp.dot(a, w[...], preferred_element_type=jnp.float32) + b[...]
    r = jax.nn.sigmoid(pre[:, :F])
    u = jax.nn.sigmoid(pre[:, F:])
    rh_o[...] = r * h[...]
    u_o[...] = u


def _combine_body(ax0, ax1, ar0, ar1, deg, h, u, w, b, h_o, c_o):
    dinv = 1.0 / jnp.maximum(deg[0], 1.0)
    a = jnp.concatenate(
        [ax0[0] * dinv, ax1[0] * dinv, ar0[0] * dinv, ar1[0] * dinv], axis=1)
    cc = jnp.tanh(jnp.dot(a, w[...], preferred_element_type=jnp.float32) + b[...])
    uu = u[...]
    h_o[...] = uu * h[...] + (1.0 - uu) * cc
    c_o[...] = cc


def _half_spec(which):
    return pl.BlockSpec((1, BT, FH), lambda i, w=which: (w, i, 0))


def _row_spec():
    return pl.BlockSpec((BT, F), lambda i: (i, 0))


_gates = pl.pallas_call(
    _gates_body,
    grid=(N // BT,),
    in_specs=[
        _half_spec(0), _half_spec(1), _half_spec(0), _half_spec(1), _half_spec(0),
        _row_spec(),
        pl.BlockSpec((2 * F, 2 * F), lambda i: (0, 0)),
        pl.BlockSpec((1, 2 * F), lambda i: (0, 0)),
    ],
    out_specs=[_row_spec(), _row_spec()],
    out_shape=[jax.ShapeDtypeStruct((N, F), jnp.float32)] * 2,
)

_combine = pl.pallas_call(
    _combine_body,
    grid=(N // BT,),
    in_specs=[
        _half_spec(0), _half_spec(1), _half_spec(0), _half_spec(1), _half_spec(0),
        _row_spec(), _row_spec(),
        pl.BlockSpec((2 * F, F), lambda i: (0, 0)),
        pl.BlockSpec((1, F), lambda i: (0, 0)),
    ],
    out_specs=[_row_spec(), _row_spec()],
    out_shape=[jax.ShapeDtypeStruct((N, F), jnp.float32)] * 2,
)


def _proj_body(z, w, b, o):
    o[...] = jnp.dot(z[...], w[...], preferred_element_type=jnp.float32) + b[...]


_proj = pl.pallas_call(
    _proj_body,
    grid=(SEQ * N // BT,),
    in_specs=[
        pl.BlockSpec((BT, F), lambda i: (i, 0)),
        pl.BlockSpec((F, F), lambda i: (0, 0)),
        pl.BlockSpec((1, F), lambda i: (0, 0)),
    ],
    out_specs=pl.BlockSpec((BT, F), lambda i: (i, 0)),
    out_shape=jax.ShapeDtypeStruct((SEQ * N, F), jnp.float32),
)


def kernel(x, edge_index, hidden_states, cell_states, params):
    src = edge_index[0].astype(jnp.int32)
    dst = edge_index[1].astype(jnp.int32)
    gidx = jnp.stack([src * 2, src * 2 + 1]).reshape(2, NSUB, EPW)
    sidx = dst.reshape(NSUB, NGRP, GRP, CH)

    deg2 = _agg(jnp.ones((N, F), jnp.float32), gidx, sidx)

    layers = params['layers']
    w_ru = [jnp.concatenate([p['Wr'], p['Wu']], axis=1) for p in layers]
    b_ru = [jnp.concatenate([p['br'], p['bu']])[None, :] for p in layers]
    w_c = [p['Wc'] for p in layers]
    b_c = [p['bc'][None, :] for p in layers]

    h = [hidden_states[j] for j in range(NLAYERS)]
    c = [None] * NLAYERS
    outs = []
    for i in range(SEQ):
        inp = x[i]
        for j in range(NLAYERS):
            aggx = _agg(inp, gidx, sidx)
            aggh = _agg(h[j], gidx, sidx)
            rh, u = _gates(aggx, aggx, aggh, aggh, deg2, h[j], w_ru[j], b_ru[j])
            aggr = _agg(rh, gidx, sidx)
            hnew, cnew = _combine(aggx, aggx, aggr, aggr, deg2, h[j], u,
                                  w_c[j], b_c[j])
            h[j] = hnew
            c[j] = cnew
            inp = hnew
        outs.append(inp)

    z = jnp.stack(outs, 0).reshape(SEQ * N, F)
    out = _proj(z, params['Wo'], params['bo'][None, :]).reshape(SEQ, N, F)
    return out, jnp.stack(h, 0), jnp.stack(c, 0)


# speedup vs baseline: 9.6132x; 1.0003x over previous
import functools

import jax
import jax.numpy as jnp
from jax import lax
from jax.experimental import pallas as pl
from jax.experimental.pallas import tpu as pltpu
from jax.experimental.pallas import tpu_sc as plsc

N = 10000
E = 160000
F = 256
FH = 128
NSUB = 16
SEQ = 6
NLAYERS = 2

EPW = E // NSUB
CH = 80
NBUF = 3
NCH = EPW // CH
WB = 80
NWB = N // WB
WBROUNDS = -(-NWB // NSUB)
GRP = 25
NGRP = NCH // GRP


_sc_mesh = plsc.VectorSubcoreMesh(core_axis_name="c", subcore_axis_name="s")


@functools.partial(
    pl.kernel,
    mesh=_sc_mesh,
    out_type=jax.ShapeDtypeStruct((2, N, FH), jnp.float32),
    scratch_types=(
        [pltpu.VMEM((EPW,), jnp.int32)]
        + [pltpu.VMEM((2, GRP, CH), jnp.int32)]
        + [pltpu.VMEM((CH, FH), jnp.float32) for _ in range(NBUF)]
        + [pltpu.VMEM_SHARED((N, FH), jnp.float32)]
        + [pltpu.SemaphoreType.DMA for _ in range(2 * NBUF + 1)]
    ),
)
def _sc_agg(feat2, gidx, sidx, out,
            srcb, dstb, rb0, rb1, rb2,
            acc, sg0, sg1, sg2, ss0, ss1, ss2, si):
    c = lax.axis_index("c")
    s = lax.axis_index("s")
    rbs = (rb0, rb1, rb2)
    sgs = (sg0, sg1, sg2)
    sss = (ss0, ss1, ss2)

    pltpu.sync_copy(gidx.at[c, s], srcb)
    pltpu.sync_copy(sidx.at[s, 0], dstb.at[0])

    zv = jnp.zeros((16,), jnp.float32)

    def zloop(r, carry):
        for jj in range(FH // 16):
            rb0[r, pl.ds(jj * 16, 16)] = zv
        return carry

    lax.fori_loop(0, WB, zloop, 0)
    for t in range(WBROUNDS):
        k = s + t * NSUB

        @pl.when(k < NWB)
        def _():
            pltpu.sync_copy(rb0, acc.at[pl.ds(pl.multiple_of(k * WB, 8), WB)])

    plsc.subcore_barrier()

    def grow(k):
        return acc.at[dstb.at[lax.rem(lax.div(k, GRP), 2), lax.rem(k, GRP)]]

    def fire_gather(k, j):
        idx = srcb.at[pl.ds(pl.multiple_of(k * CH, 8), CH)]
        pltpu.make_async_copy(feat2.at[idx], rbs[j], sgs[j]).start()

    def wait_gather(k, j):
        idx = srcb.at[pl.ds(pl.multiple_of(k * CH, 8), CH)]
        pltpu.make_async_copy(feat2.at[idx], rbs[j], sgs[j]).wait()

    def step(k, j):
        jn = (j + 1) % NBUF
        kg = lax.rem(k, GRP)

        @pl.when(jnp.logical_and(kg == 0, k > 0))
        def _():
            pltpu.make_async_copy(sidx.at[s, 0], dstb.at[0], si).wait()

        @pl.when(k >= 2)
        def _():
            pltpu.make_async_copy(rbs[jn], grow(k), sss[jn]).wait()

        @pl.when(jnp.logical_and(kg == 2, k < (NGRP - 1) * GRP))
        def _():
            g1 = lax.div(k, GRP) + 1
            pltpu.make_async_copy(sidx.at[s, g1], dstb.at[lax.rem(g1, 2)],
                                  si).start()

        @pl.when(k + 1 < NCH)
        def _():
            fire_gather(k + 1, jn)

        wait_gather(k, j)
        pltpu.async_copy(rbs[j], grow(k), sss[j], add=True)

    fire_gather(0, 0)

    def mainblk(kb, carry):
        for j in range(NBUF):
            step(kb * NBUF + j, j)
        return carry

    lax.fori_loop(0, NCH // NBUF, mainblk, 0)
    pltpu.make_async_copy(rbs[1], grow(NCH - 4), sss[1]).wait()
    fire_gather(NCH - 1, 1)
    wait_gather(NCH - 2, 0)
    pltpu.async_copy(rbs[0], grow(NCH - 2), sss[0], add=True)
    pltpu.make_async_copy(rbs[2], grow(NCH - 3), sss[2]).wait()
    wait_gather(NCH - 1, 1)
    pltpu.async_copy(rbs[1], grow(NCH - 1), sss[1], add=True)
    pltpu.make_async_copy(rbs[0], grow(NCH - 2), sss[0]).wait()
    pltpu.make_async_copy(rbs[1], grow(NCH - 1), sss[1]).wait()

    plsc.subcore_barrier()
    for t in range(WBROUNDS):
        k = s + t * NSUB

        @pl.when(k < NWB)
        def _():
            r0 = pl.multiple_of(k * WB, 8)
            pltpu.sync_copy(acc.at[pl.ds(r0, WB)], out.at[c, pl.ds(r0, WB)])


def _agg(feat, gidx, sidx):
    return _sc_agg(feat.reshape(2 * N, FH), gidx, sidx)


BT = 2000


def _gates_body(ax0, ax1, ah0, ah1, deg, h, w, b, rh_o, u_o):
    dinv = 1.0 / jnp.maximum(deg[0], 1.0)
    a = jnp.concatenate(
        [ax0[0] * dinv, ax1[0] * dinv, ah0[0] * dinv, ah1[0] * dinv], axis=1)
    pre = jnp.dot(a, w[...], preferred_element_type=jnp.float32) + b[...]
    r = jax.nn.sigmoid(pre[:, :F])
    u = jax.nn.sigmoid(pre[:, F:])
    rh_o[...] = r * h[...]
    u_o[...] = u


def _combine_body(ax0, ax1, ar0, ar1, deg, h, u, w, b, h_o, c_o):
    dinv = 1.0 / jnp.maximum(deg[0], 1.0)
    a = jnp.concatenate(
        [ax0[0] * dinv, ax1[0] * dinv, ar0[0] * dinv, ar1[0] * dinv], axis=1)
    cc = jnp.tanh(jnp.dot(a, w[...], preferred_element_type=jnp.float32) + b[...])
    uu = u[...]
    h_o[...] = uu * h[...] + (1.0 - uu) * cc
    c_o[...] = cc


def _half_spec(which):
    return pl.BlockSpec((1, BT, FH), lambda i, w=which: (w, i, 0))


def _row_spec():
    return pl.BlockSpec((BT, F), lambda i: (i, 0))


_gates = pl.pallas_call(
    _gates_body,
    grid=(N // BT,),
    in_specs=[
        _half_spec(0), _half_spec(1), _half_spec(0), _half_spec(1), _half_spec(0),
        _row_spec(),
        pl.BlockSpec((2 * F, 2 * F), lambda i: (0, 0)),
        pl.BlockSpec((1, 2 * F), lambda i: (0, 0)),
    ],
    out_specs=[_row_spec(), _row_spec()],
    out_shape=[jax.ShapeDtypeStruct((N, F), jnp.float32)] * 2,
)

_combine = pl.pallas_call(
    _combine_body,
    grid=(N // BT,),
    in_specs=[
        _half_spec(0), _half_spec(1), _half_spec(0), _half_spec(1), _half_spec(0),
        _row_spec(), _row_spec(),
        pl.BlockSpec((2 * F, F), lambda i: (0, 0)),
        pl.BlockSpec((1, F), lambda i: (0, 0)),
    ],
    out_specs=[_row_spec(), _row_spec()],
    out_shape=[jax.ShapeDtypeStruct((N, F), jnp.float32)] * 2,
)


def _proj_body(z, w, b, o):
    o[...] = jnp.dot(z[...], w[...], preferred_element_type=jnp.float32) + b[...]


_proj = pl.pallas_call(
    _proj_body,
    grid=(SEQ * N // BT,),
    in_specs=[
        pl.BlockSpec((BT, F), lambda i: (i, 0)),
        pl.BlockSpec((F, F), lambda i: (0, 0)),
        pl.BlockSpec((1, F), lambda i: (0, 0)),
    ],
    out_specs=pl.BlockSpec((BT, F), lambda i: (i, 0)),
    out_shape=jax.ShapeDtypeStruct((SEQ * N, F), jnp.float32),
)


def kernel(x, edge_index, hidden_states, cell_states, params):
    src = edge_index[0].astype(jnp.int32)
    dst = edge_index[1].astype(jnp.int32)
    gidx = jnp.stack([src * 2, src * 2 + 1]).reshape(2, NSUB, EPW)
    sidx = dst.reshape(NSUB, NGRP, GRP, CH)

    deg2 = _agg(jnp.ones((N, F), jnp.float32), gidx, sidx)

    layers = params['layers']
    w_ru = [jnp.concatenate([p['Wr'], p['Wu']], axis=1) for p in layers]
    b_ru = [jnp.concatenate([p['br'], p['bu']])[None, :] for p in layers]
    w_c = [p['Wc'] for p in layers]
    b_c = [p['bc'][None, :] for p in layers]

    h = [hidden_states[j] for j in range(NLAYERS)]
    c = [None] * NLAYERS
    outs = []
    aggx0 = [_agg(x[i], gidx, sidx) for i in range(SEQ)]
    for i in range(SEQ):
        aggx = aggx0[i]
        aggh1 = _agg(h[1], gidx, sidx)
        for j in range(NLAYERS):
            aggh = aggh1 if j == 1 else _agg(h[j], gidx, sidx)
            rh, u = _gates(aggx, aggx, aggh, aggh, deg2, h[j], w_ru[j], b_ru[j])
            aggr = _agg(rh, gidx, sidx)
            hnew, cnew = _combine(aggx, aggx, aggr, aggr, deg2, h[j], u,
                                  w_c[j], b_c[j])
            h[j] = hnew
            c[j] = cnew
            if j + 1 < NLAYERS:
                aggx = _agg(hnew, gidx, sidx)
        outs.append(h[NLAYERS - 1])

    z = jnp.stack(outs, 0).reshape(SEQ * N, F)
    out = _proj(z, params['Wo'], params['bo'][None, :]).reshape(SEQ, N, F)
    return out, jnp.stack(h, 0), jnp.stack(c, 0)


# speedup vs baseline: 9.6136x; 1.0000x over previous
import functools

import jax
import jax.numpy as jnp
from jax import lax
from jax.experimental import pallas as pl
from jax.experimental.pallas import tpu as pltpu
from jax.experimental.pallas import tpu_sc as plsc

N = 10000
E = 160000
F = 256
FH = 128
NSUB = 16
SEQ = 6
NLAYERS = 2

EPW = E // NSUB
CH = 80
NBUF = 3
NCH = EPW // CH
WB = 80
NWB = N // WB
WBROUNDS = -(-NWB // NSUB)
GRP = 25
NGRP = NCH // GRP


_sc_mesh = plsc.VectorSubcoreMesh(core_axis_name="c", subcore_axis_name="s")


@functools.partial(
    pl.kernel,
    mesh=_sc_mesh,
    out_type=jax.ShapeDtypeStruct((2, N, FH), jnp.float32),
    scratch_types=(
        [pltpu.VMEM((EPW,), jnp.int32)]
        + [pltpu.VMEM((2, GRP, CH), jnp.int32)]
        + [pltpu.VMEM((CH, FH), jnp.float32) for _ in range(NBUF)]
        + [pltpu.VMEM_SHARED((N, FH), jnp.float32)]
        + [pltpu.SemaphoreType.DMA for _ in range(2 * NBUF + 1)]
    ),
)
def _sc_agg(feat2, gidx, sidx, out,
            srcb, dstb, rb0, rb1, rb2,
            acc, sg0, sg1, sg2, ss0, ss1, ss2, si):
    c = lax.axis_index("c")
    s = lax.axis_index("s")
    rbs = (rb0, rb1, rb2)
    sgs = (sg0, sg1, sg2)
    sss = (ss0, ss1, ss2)

    pltpu.sync_copy(gidx.at[c, s], srcb)
    pltpu.sync_copy(sidx.at[s, 0], dstb.at[0])

    zv = jnp.zeros((16,), jnp.float32)

    def zloop(r, carry):
        for jj in range(FH // 16):
            rb0[r, pl.ds(jj * 16, 16)] = zv
        return carry

    lax.fori_loop(0, WB, zloop, 0)
    for t in range(WBROUNDS):
        k = s + t * NSUB

        @pl.when(k < NWB)
        def _():
            pltpu.sync_copy(rb0, acc.at[pl.ds(pl.multiple_of(k * WB, 8), WB)])

    plsc.subcore_barrier()

    def grow(k):
        return acc.at[dstb.at[lax.rem(lax.div(k, GRP), 2), lax.rem(k, GRP)]]

    def fire_gather(k, j):
        idx = srcb.at[pl.ds(pl.multiple_of(k * CH, 8), CH)]
        pltpu.make_async_copy(feat2.at[idx], rbs[j], sgs[j]).start()

    def wait_gather(k, j):
        idx = srcb.at[pl.ds(pl.multiple_of(k * CH, 8), CH)]
        pltpu.make_async_copy(feat2.at[idx], rbs[j], sgs[j]).wait()

    def step(k, j):
        jn = (j + 1) % NBUF
        kg = lax.rem(k, GRP)

        @pl.when(jnp.logical_and(kg == 0, k > 0))
        def _():
            pltpu.make_async_copy(sidx.at[s, 0], dstb.at[0], si).wait()

        @pl.when(k >= 2)
        def _():
            pltpu.make_async_copy(rbs[jn], grow(k), sss[jn]).wait()

        @pl.when(jnp.logical_and(kg == 2, k < (NGRP - 1) * GRP))
        def _():
            g1 = lax.div(k, GRP) + 1
            pltpu.make_async_copy(sidx.at[s, g1], dstb.at[lax.rem(g1, 2)],
                                  si).start()

        @pl.when(k + 1 < NCH)
        def _():
            fire_gather(k + 1, jn)

        wait_gather(k, j)
        pltpu.async_copy(rbs[j], grow(k), sss[j], add=True)

    fire_gather(0, 0)

    def mainblk(kb, carry):
        for j in range(NBUF):
            step(kb * NBUF + j, j)
        return carry

    lax.fori_loop(0, NCH // NBUF, mainblk, 0)
    pltpu.make_async_copy(rbs[1], grow(NCH - 4), sss[1]).wait()
    fire_gather(NCH - 1, 1)
    wait_gather(NCH - 2, 0)
    pltpu.async_copy(rbs[0], grow(NCH - 2), sss[0], add=True)
    pltpu.make_async_copy(rbs[2], grow(NCH - 3), sss[2]).wait()
    wait_gather(NCH - 1, 1)
    pltpu.async_copy(rbs[1], grow(NCH - 1), sss[1], add=True)
    pltpu.make_async_copy(rbs[0], grow(NCH - 2), sss[0]).wait()
    pltpu.make_async_copy(rbs[1], grow(NCH - 1), sss[1]).wait()

    plsc.subcore_barrier()
    for t in range(WBROUNDS):
        k = s + t * NSUB

        @pl.when(k < NWB)
        def _():
            r0 = pl.multiple_of(k * WB, 8)
            pltpu.sync_copy(acc.at[pl.ds(r0, WB)], out.at[c, pl.ds(r0, WB)])


def _agg(feat, gidx, sidx):
    return _sc_agg(feat.reshape(2 * N, FH), gidx, sidx)


BT = 2000


def _gates_body(ax0, ax1, ah0, ah1, deg, h, w, b, rh_o, u_o):
    dinv = 1.0 / jnp.maximum(deg[0], 1.0)
    a = jnp.concatenate(
        [ax0[0] * dinv, ax1[0] * dinv, ah0[0] * dinv, ah1[0] * dinv], axis=1)
    pre = jnp.dot(a, w[...], preferred_element_type=jnp.float32) + b[...]
    r = jax.nn.sigmoid(pre[:, :F])
    u = jax.nn.sigmoid(pre[:, F:])
    rh_o[...] = r * h[...]
    u_o[...] = u


def _combine_body(ax0, ax1, ar0, ar1, deg, h, u, w, b, h_o, c_o):
    dinv = 1.0 / jnp.maximum(deg[0], 1.0)
    a = jnp.concatenate(
        [ax0[0] * dinv, ax1[0] * dinv, ar0[0] * dinv, ar1[0] * dinv], axis=1)
    cc = jnp.tanh(jnp.dot(a, w[...], preferred_element_type=jnp.float32) + b[...])
    uu = u[...]
    h_o[...] = uu * h[...] + (1.0 - uu) * cc
    c_o[...] = cc


def _half_spec(which):
    return pl.BlockSpec((1, BT, FH), lambda i, w=which: (w, i, 0))


def _row_spec():
    return pl.BlockSpec((BT, F), lambda i: (i, 0))


_gates = pl.pallas_call(
    _gates_body,
    grid=(N // BT,),
    in_specs=[
        _half_spec(0), _half_spec(1), _half_spec(0), _half_spec(1), _half_spec(0),
        _row_spec(),
        pl.BlockSpec((2 * F, 2 * F), lambda i: (0, 0)),
        pl.BlockSpec((1, 2 * F), lambda i: (0, 0)),
    ],
    out_specs=[_row_spec(), _row_spec()],
    out_shape=[jax.ShapeDtypeStruct((N, F), jnp.float32)] * 2,
)

_combine = pl.pallas_call(
    _combine_body,
    grid=(N // BT,),
    in_specs=[
        _half_spec(0), _half_spec(1), _half_spec(0), _half_spec(1), _half_spec(0),
        _row_spec(), _row_spec(),
        pl.BlockSpec((2 * F, F), lambda i: (0, 0)),
        pl.BlockSpec((1, F), lambda i: (0, 0)),
    ],
    out_specs=[_row_spec(), _row_spec()],
    out_shape=[jax.ShapeDtypeStruct((N, F), jnp.float32)] * 2,
)


def _proj_body(z, w, b, o):
    o[...] = jnp.dot(z[...], w[...], preferred_element_type=jnp.float32) + b[...]


_proj = pl.pallas_call(
    _proj_body,
    grid=(SEQ * N // BT,),
    in_specs=[
        pl.BlockSpec((BT, F), lambda i: (i, 0)),
        pl.BlockSpec((F, F), lambda i: (0, 0)),
        pl.BlockSpec((1, F), lambda i: (0, 0)),
    ],
    out_specs=pl.BlockSpec((BT, F), lambda i: (i, 0)),
    out_shape=jax.ShapeDtypeStruct((SEQ * N, F), jnp.float32),
)


def kernel(x, edge_index, hidden_states, cell_states, params):
    src = edge_index[0].astype(jnp.int32)
    dst = edge_index[1].astype(jnp.int32)
    gidx = jnp.stack([src * 2, src * 2 + 1]).reshape(2, NSUB, EPW)
    sidx = dst.reshape(NSUB, NGRP, GRP, CH)

    deg2 = _agg(jnp.ones((N, F), jnp.float32), gidx, sidx)

    layers = params['layers']
    w_ru = [jnp.concatenate([p['Wr'], p['Wu']], axis=1) for p in layers]
    b_ru = [jnp.concatenate([p['br'], p['bu']])[None, :] for p in layers]
    w_c = [p['Wc'] for p in layers]
    b_c = [p['bc'][None, :] for p in layers]

    h = [hidden_states[j] for j in range(NLAYERS)]
    c = [None] * NLAYERS
    outs = []
    aggx0 = [_agg(x[i], gidx, sidx) for i in range(SEQ)]
    prev_aggx1 = None
    for i in range(SEQ):
        aggx = aggx0[i]
        aggh1 = _agg(h[1], gidx, sidx)
        for j in range(NLAYERS):
            if j == 0:
                aggh = prev_aggx1 if i > 0 else _agg(h[0], gidx, sidx)
            else:
                aggh = aggh1
            rh, u = _gates(aggx, aggx, aggh, aggh, deg2, h[j], w_ru[j], b_ru[j])
            aggr = _agg(rh, gidx, sidx)
            hnew, cnew = _combine(aggx, aggx, aggr, aggr, deg2, h[j], u,
                                  w_c[j], b_c[j])
            h[j] = hnew
            c[j] = cnew
            if j + 1 < NLAYERS:
                aggx = _agg(hnew, gidx, sidx)
                prev_aggx1 = aggx
        outs.append(h[NLAYERS - 1])

    z = jnp.stack(outs, 0).reshape(SEQ * N, F)
    out = _proj(z, params['Wo'], params['bo'][None, :]).reshape(SEQ, N, F)
    return out, jnp.stack(h, 0), jnp.stack(c, 0)
